# baseline (speedup 1.0000x reference)
import jax
import jax.numpy as jnp
from jax.experimental import pallas as pl


def _xw_kernel(x_ref, w_ref, o_ref):
    o_ref[...] = jnp.dot(x_ref[...], w_ref[...],
                         preferred_element_type=jnp.float32)


def _pass1_kernel(adj_ref, a_ref, b1_ref, w2_ref, g_ref):
    h = jnp.dot(adj_ref[...], a_ref[...], preferred_element_type=jnp.float32)
    h = jnp.maximum(h + b1_ref[...], 0.0)
    g_ref[...] = jnp.dot(h, w2_ref[...], preferred_element_type=jnp.float32)


def _pass2_kernel(adj_ref, g_ref, b2_ref, o_ref):
    z = jnp.dot(adj_ref[...], g_ref[...], preferred_element_type=jnp.float32)
    z = jnp.maximum(z + b2_ref[...], 0.0)
    m = jnp.max(z, axis=1, keepdims=True)
    s = z - m
    lse = jnp.log(jnp.sum(jnp.exp(s), axis=1, keepdims=True))
    o_ref[...] = s - lse


def kernel(x, adj, W1, b1, W2, b2):
    n, d_in = x.shape
    hid = W1.shape[1]
    classes = W2.shape[1]
    b1r = b1.reshape(1, hid)
    b2r = b2.reshape(1, classes)

    a = pl.pallas_call(
        _xw_kernel,
        out_shape=jax.ShapeDtypeStruct((n, hid), jnp.float32),
    )(x, W1)

    br = 400
    grid = (n // br,)

    g = pl.pallas_call(
        _pass1_kernel,
        grid=grid,
        in_specs=[
            pl.BlockSpec((br, n), lambda i: (i, 0)),
            pl.BlockSpec((n, hid), lambda i: (0, 0)),
            pl.BlockSpec((1, hid), lambda i: (0, 0)),
            pl.BlockSpec((hid, classes), lambda i: (0, 0)),
        ],
        out_specs=pl.BlockSpec((br, classes), lambda i: (i, 0)),
        out_shape=jax.ShapeDtypeStruct((n, classes), jnp.float32),
    )(adj, a, b1r, W2)

    out = pl.pallas_call(
        _pass2_kernel,
        grid=grid,
        in_specs=[
            pl.BlockSpec((br, n), lambda i: (i, 0)),
            pl.BlockSpec((n, classes), lambda i: (0, 0)),
            pl.BlockSpec((1, classes), lambda i: (0, 0)),
        ],
        out_specs=pl.BlockSpec((br, classes), lambda i: (i, 0)),
        out_shape=jax.ShapeDtypeStruct((n, classes), jnp.float32),
    )(adj, g, b2r)
    return out
